# baseline (speedup 1.0000x reference)
import numpy as np
import jax
import jax.numpy as jnp
from jax.experimental import pallas as pl
from jax.experimental.pallas import tpu as pltpu

TOPK = 4
BLK = 8192
CH = 1024
BIGI = np.int32(2**30)
NEG = np.float32(-np.inf)
TINY = np.float32(np.finfo(np.float32).tiny)


def _select_topk(vals, idx, k=TOPK):
    out_v, out_i = [], []
    for _ in range(k):
        m = jnp.max(vals, axis=1, keepdims=True)
        sel = jnp.min(jnp.where(vals == m, idx, BIGI), axis=1, keepdims=True)
        out_v.append(m)
        out_i.append(sel)
        kill = (vals == m) & (idx == sel)
        vals = jnp.where(kill, NEG, vals)
    return jnp.concatenate(out_v, axis=1), jnp.concatenate(out_i, axis=1)


def _threefry_bits(p_u32):
    k0 = jnp.uint32(0)
    k1 = jnp.uint32(42)
    k2 = k0 ^ k1 ^ jnp.uint32(0x1BD11BDA)
    ks = [k0, k1, k2]
    rot0 = (13, 15, 26, 6)
    rot1 = (17, 29, 16, 24)

    def rotl(x, d):
        return (x << jnp.uint32(d)) | (x >> jnp.uint32(32 - d))

    x0 = jnp.zeros_like(p_u32) + k0
    x1 = p_u32 + k1
    for r in range(5):
        for d in (rot0 if r % 2 == 0 else rot1):
            x0 = x0 + x1
            x1 = rotl(x1, d) ^ x0
        x0 = x0 + ks[(r + 1) % 3]
        x1 = x1 + ks[(r + 2) % 3] + jnp.uint32(r + 1)
    return x0 ^ x1


def _gumbel_at(p_i32):
    bits = _threefry_bits(p_i32.astype(jnp.uint32))
    fb = (bits >> jnp.uint32(9)) | jnp.uint32(0x3F800000)
    f = jax.lax.bitcast_convert_type(fb, jnp.float32) - jnp.float32(1.0)
    u = jnp.maximum(TINY, f + TINY)
    return -jnp.log(-jnp.log(u))


def _make_kernel(B, N, nblk):
    def body(x_ref, out_ref, s_ref, sv, si, srv):
        ph = pl.program_id(0)
        step = pl.program_id(1)

        @pl.when((ph == 0) & (step == 0))
        def _init():
            sv[...] = jnp.full((B, TOPK), NEG, jnp.float32)
            si[...] = jnp.full((B, TOPK), BIGI, jnp.int32)

        col = jax.lax.broadcasted_iota(jnp.int32, (B, BLK), 1) + step * BLK

        @pl.when(ph == 0)
        def _scan():
            for c in range(BLK // CH):
                xc = x_ref[:, pl.ds(c * CH, CH)]
                trig = jnp.any(xc > sv[:, 3:4])

                @pl.when(trig)
                def _merge_chunk(xc=xc, c=c):
                    colc = (
                        jax.lax.broadcasted_iota(jnp.int32, (B, CH), 1)
                        + step * BLK
                        + c * CH
                    )
                    xm = jnp.where(colc < N, xc, NEG)
                    bv, bi = _select_topk(xm, colc)
                    cv = jnp.concatenate([sv[...], bv], axis=1)
                    ci = jnp.concatenate([si[...], bi], axis=1)
                    nv, ni = _select_topk(cv, ci)
                    sv[...] = nv
                    si[...] = ni

            @pl.when(step == nblk - 1)
            def _finalize():
                v = sv[...]
                ix = si[...]
                rv = v / jnp.sum(v, axis=1, keepdims=True)
                srv[...] = rv
                p = ix + jax.lax.broadcasted_iota(jnp.int32, (B, TOPK), 0) * N
                score = jnp.log(rv + jnp.float32(1e-20)) + _gumbel_at(p)
                m = jnp.max(score, axis=1, keepdims=True)
                j = jax.lax.broadcasted_iota(jnp.int32, (B, TOPK), 1)
                jsel = jnp.min(
                    jnp.where(score == m, j, BIGI), axis=1, keepdims=True
                )
                s_ref[...] = jnp.sum(
                    jnp.where(j == jsel, ix, 0), axis=1, keepdims=True
                )

        @pl.when(ph == 1)
        def _write():
            hit = jnp.any(si[...] // BLK == step)

            @pl.when(hit)
            def _scatter_block():
                acc = jnp.zeros((B, BLK), jnp.float32)
                for jj in range(TOPK):
                    cj = si[:, pl.ds(jj, 1)]
                    vj = srv[:, pl.ds(jj, 1)]
                    acc = jnp.where(col == cj, vj, acc)
                out_ref[...] = acc

            @pl.when(jnp.logical_not(hit))
            def _zeros_block():
                out_ref[...] = jnp.zeros((B, BLK), jnp.float32)

    return body


def kernel(softmax):
    B, N = softmax.shape
    nblk = (N + BLK - 1) // BLK

    renorm, s2d = pl.pallas_call(
        _make_kernel(B, N, nblk),
        grid=(2, nblk),
        in_specs=[pl.BlockSpec((B, BLK), lambda ph, i: (0, i * (1 - ph)))],
        out_specs=[
            pl.BlockSpec((B, BLK), lambda ph, i: (0, i * ph)),
            pl.BlockSpec((B, 1), lambda ph, i: (0, 0)),
        ],
        out_shape=[
            jax.ShapeDtypeStruct((B, N), jnp.float32),
            jax.ShapeDtypeStruct((B, 1), jnp.int32),
        ],
        scratch_shapes=[
            pltpu.VMEM((B, TOPK), jnp.float32),
            pltpu.VMEM((B, TOPK), jnp.int32),
            pltpu.VMEM((B, TOPK), jnp.float32),
        ],
    )(softmax)

    return renorm, s2d.reshape(B)


# speedup vs baseline: 1.9862x; 1.9862x over previous
import numpy as np
import jax
import jax.numpy as jnp
from jax.experimental import pallas as pl
from jax.experimental.pallas import tpu as pltpu

TOPK = 4
BLK = 8192
RG = 8
NBPAD = 128
BIGI = np.int32(2**30)
NEG = np.float32(-np.inf)
TINY = np.float32(np.finfo(np.float32).tiny)


def _select_topk(vals, idx, k=TOPK):
    out_v, out_i = [], []
    for _ in range(k):
        m = jnp.max(vals, axis=1, keepdims=True)
        sel = jnp.min(jnp.where(vals == m, idx, BIGI), axis=1, keepdims=True)
        out_v.append(m)
        out_i.append(sel)
        vals = jnp.where(idx == sel, NEG, vals)
    return jnp.concatenate(out_v, axis=1), jnp.concatenate(out_i, axis=1)


def _threefry_bits(p_u32):
    k0 = jnp.uint32(0)
    k1 = jnp.uint32(42)
    k2 = k0 ^ k1 ^ jnp.uint32(0x1BD11BDA)
    ks = [k0, k1, k2]
    rot0 = (13, 15, 26, 6)
    rot1 = (17, 29, 16, 24)

    def rotl(x, d):
        return (x << jnp.uint32(d)) | (x >> jnp.uint32(32 - d))

    x0 = jnp.zeros_like(p_u32) + k0
    x1 = p_u32 + k1
    for r in range(5):
        for d in (rot0 if r % 2 == 0 else rot1):
            x0 = x0 + x1
            x1 = rotl(x1, d) ^ x0
        x0 = x0 + ks[(r + 1) % 3]
        x1 = x1 + ks[(r + 2) % 3] + jnp.uint32(r + 1)
    return x0 ^ x1


def _gumbel_at(p_i32):
    bits = _threefry_bits(p_i32.astype(jnp.uint32))
    fb = (bits >> jnp.uint32(9)) | jnp.uint32(0x3F800000)
    f = jax.lax.bitcast_convert_type(fb, jnp.float32) - jnp.float32(1.0)
    u = jnp.maximum(TINY, f + TINY)
    return -jnp.log(-jnp.log(u))


def _make_blockmax_kernel(B, N, nblk):
    def body(x_ref, bb_ref, bm):
        step = pl.program_id(0)

        @pl.when(step == 0)
        def _init():
            bm[...] = jnp.full((B, NBPAD), NEG, jnp.float32)

        lane = jax.lax.broadcasted_iota(jnp.int32, (B, NBPAD), 1)
        m = jnp.max(x_ref[...], axis=1, keepdims=True)
        bm[...] = jnp.where(lane == step, jnp.broadcast_to(m, (B, NBPAD)), bm[...])

        @pl.when(step == nblk - 1)
        def _fix_tail_and_select():
            col = jax.lax.broadcasted_iota(jnp.int32, (B, BLK), 1) + step * BLK
            xm = jnp.where(col < N, x_ref[...], NEG)
            mt = jnp.max(xm, axis=1, keepdims=True)
            bm[...] = jnp.where(
                lane == step, jnp.broadcast_to(mt, (B, NBPAD)), bm[...]
            )
            bids = jax.lax.broadcasted_iota(jnp.int32, (B, NBPAD), 1)
            _, bb = _select_topk(bm[...], bids)
            bb_ref[...] = bb

    return body


def _make_main_kernel(B, N, nblk):
    def body(bb_pref, x_ref, out_ref, s_ref, sv, si, srv):
        ph = pl.program_id(0)
        s = pl.program_id(1)

        @pl.when((ph == 0) & (s == 0))
        def _init():
            sv[...] = jnp.full((B, TOPK), NEG, jnp.float32)
            si[...] = jnp.full((B, TOPK), BIGI, jnp.int32)

        @pl.when(ph == 0)
        def _scan_candidate():
            r = s // TOPK
            bid = bb_pref[s]
            colc = (
                jax.lax.broadcasted_iota(jnp.int32, (RG, BLK), 1) + bid * BLK
            )
            sub = jax.lax.broadcasted_iota(jnp.int32, (RG, BLK), 0)
            xm = jnp.where(
                (colc < N) & (sub == r % RG), x_ref[...], NEG
            )
            bv, bi = _select_topk(xm, colc)
            subk = jax.lax.broadcasted_iota(jnp.int32, (RG, TOPK), 0)
            pickrow = subk == r % RG
            bv1 = jnp.max(jnp.where(pickrow, bv, NEG), axis=0, keepdims=True)
            bi1 = jnp.sum(jnp.where(pickrow, bi, 0), axis=0, keepdims=True)
            rmask = jax.lax.broadcasted_iota(jnp.int32, (B, TOPK), 0) == r
            cv = jnp.concatenate(
                [sv[...], jnp.broadcast_to(bv1, (B, TOPK))], axis=1
            )
            ci = jnp.concatenate(
                [si[...], jnp.broadcast_to(bi1, (B, TOPK))], axis=1
            )
            nv, ni = _select_topk(cv, ci)
            sv[...] = jnp.where(rmask, nv, sv[...])
            si[...] = jnp.where(rmask, ni, si[...])

        @pl.when(ph == 1)
        def _write():
            @pl.when(s == 0)
            def _finalize():
                v = sv[...]
                ix = si[...]
                rv = v / jnp.sum(v, axis=1, keepdims=True)
                srv[...] = rv
                p = ix + jax.lax.broadcasted_iota(jnp.int32, (B, TOPK), 0) * N
                score = jnp.log(rv + jnp.float32(1e-20)) + _gumbel_at(p)
                m = jnp.max(score, axis=1, keepdims=True)
                j = jax.lax.broadcasted_iota(jnp.int32, (B, TOPK), 1)
                jsel = jnp.min(
                    jnp.where(score == m, j, BIGI), axis=1, keepdims=True
                )
                s_ref[...] = jnp.sum(
                    jnp.where(j == jsel, ix, 0), axis=1, keepdims=True
                )

            bidx = jnp.minimum(s, nblk - 1)
            col = jax.lax.broadcasted_iota(jnp.int32, (B, BLK), 1) + bidx * BLK
            hit = jnp.any(si[...] // BLK == bidx)

            @pl.when(hit)
            def _scatter_block():
                acc = jnp.zeros((B, BLK), jnp.float32)
                for jj in range(TOPK):
                    cj = si[:, pl.ds(jj, 1)]
                    vj = srv[:, pl.ds(jj, 1)]
                    acc = jnp.where(col == cj, vj, acc)
                out_ref[...] = acc

            @pl.when(jnp.logical_not(hit))
            def _zeros_block():
                out_ref[...] = jnp.zeros((B, BLK), jnp.float32)

    return body


def kernel(softmax):
    B, N = softmax.shape
    nblk = (N + BLK - 1) // BLK
    G = max(B * TOPK, nblk)

    bb = pl.pallas_call(
        _make_blockmax_kernel(B, N, nblk),
        grid=(nblk,),
        in_specs=[pl.BlockSpec((B, BLK), lambda i: (0, i))],
        out_specs=pl.BlockSpec((B, TOPK), lambda i: (0, 0)),
        out_shape=jax.ShapeDtypeStruct((B, TOPK), jnp.int32),
        scratch_shapes=[pltpu.VMEM((B, NBPAD), jnp.float32)],
    )(softmax)

    renorm, s2d = pl.pallas_call(
        _make_main_kernel(B, N, nblk),
        grid_spec=pltpu.PrefetchScalarGridSpec(
            num_scalar_prefetch=1,
            grid=(2, G),
            in_specs=[
                pl.BlockSpec(
                    (RG, BLK),
                    lambda ph, s, bb: (
                        (s // (TOPK * RG)) * (1 - ph),
                        bb[s] * (1 - ph),
                    ),
                ),
            ],
            out_specs=[
                pl.BlockSpec(
                    (B, BLK),
                    lambda ph, s, bb: (0, jnp.minimum(s, nblk - 1) * ph),
                ),
                pl.BlockSpec((B, 1), lambda ph, s, bb: (0, 0)),
            ],
            scratch_shapes=[
                pltpu.VMEM((B, TOPK), jnp.float32),
                pltpu.VMEM((B, TOPK), jnp.int32),
                pltpu.VMEM((B, TOPK), jnp.float32),
            ],
        ),
        out_shape=[
            jax.ShapeDtypeStruct((B, N), jnp.float32),
            jax.ShapeDtypeStruct((B, 1), jnp.int32),
        ],
    )(bb.reshape(-1), softmax)

    return renorm, s2d.reshape(B)


# speedup vs baseline: 4.7383x; 2.3856x over previous
import numpy as np
import jax
import jax.numpy as jnp
from jax.experimental import pallas as pl
from jax.experimental.pallas import tpu as pltpu

TOPK = 4
BLK = 8192
BLKF = 4 * BLK
BLKW = 4 * BLK
NBPAD = 128
BIGI = np.int32(2**30)
NEG = np.float32(-np.inf)
TINY = np.float32(np.finfo(np.float32).tiny)


def _select_topk(vals, idx, k=TOPK):
    out_v, out_i = [], []
    for _ in range(k):
        m = jnp.max(vals, axis=1, keepdims=True)
        sel = jnp.min(jnp.where(vals == m, idx, BIGI), axis=1, keepdims=True)
        out_v.append(m)
        out_i.append(sel)
        vals = jnp.where(idx == sel, NEG, vals)
    return jnp.concatenate(out_v, axis=1), jnp.concatenate(out_i, axis=1)


def _threefry_bits(p_u32):
    k0 = jnp.uint32(0)
    k1 = jnp.uint32(42)
    k2 = k0 ^ k1 ^ jnp.uint32(0x1BD11BDA)
    ks = [k0, k1, k2]
    rot0 = (13, 15, 26, 6)
    rot1 = (17, 29, 16, 24)

    def rotl(x, d):
        return (x << jnp.uint32(d)) | (x >> jnp.uint32(32 - d))

    x0 = jnp.zeros_like(p_u32) + k0
    x1 = p_u32 + k1
    for r in range(5):
        for d in (rot0 if r % 2 == 0 else rot1):
            x0 = x0 + x1
            x1 = rotl(x1, d) ^ x0
        x0 = x0 + ks[(r + 1) % 3]
        x1 = x1 + ks[(r + 2) % 3] + jnp.uint32(r + 1)
    return x0 ^ x1


def _gumbel_at(p_i32):
    bits = _threefry_bits(p_i32.astype(jnp.uint32))
    fb = (bits >> jnp.uint32(9)) | jnp.uint32(0x3F800000)
    f = jax.lax.bitcast_convert_type(fb, jnp.float32) - jnp.float32(1.0)
    u = jnp.maximum(TINY, f + TINY)
    return -jnp.log(-jnp.log(u))


def _make_blockmax_kernel(B, N, nblk, nsteps):
    nsub = BLKF // BLK

    def body(x_ref, bb_ref, bm):
        step = pl.program_id(0)

        @pl.when(step == 0)
        def _init():
            bm[...] = jnp.full((B, NBPAD), NEG, jnp.float32)

        lane = jax.lax.broadcasted_iota(jnp.int32, (B, NBPAD), 1)

        def merge(x):
            for q in range(nsub):
                k = step * nsub + q
                m = jnp.max(x[:, q * BLK:(q + 1) * BLK], axis=1, keepdims=True)
                bm[...] = jnp.where(
                    (lane == k) & (k < nblk),
                    jnp.broadcast_to(m, (B, NBPAD)),
                    bm[...],
                )

        merge(x_ref[...])

        @pl.when(step == nsteps - 1)
        def _fix_tail_and_select():
            col = (
                jax.lax.broadcasted_iota(jnp.int32, (B, BLKF), 1) + step * BLKF
            )
            merge(jnp.where(col < N, x_ref[...], NEG))
            bids = jax.lax.broadcasted_iota(jnp.int32, (B, NBPAD), 1)
            _, bb = _select_topk(bm[...], bids)
            bb_ref[...] = bb

    return body


def _make_main_kernel(B, N, nblkw):
    RG = 8

    def body(bb_pref, x0, x1, x2, x3, out_ref, s_ref, sv, si, srv):
        ph = pl.program_id(0)
        s = pl.program_id(1)

        @pl.when(ph == 0)
        def _scan_row():
            sub = jax.lax.broadcasted_iota(jnp.int32, (RG, BLK), 0)
            picked = []
            for xref in (x0, x1, x2, x3):
                xm = jnp.where(sub == s % RG, xref[...], NEG)
                picked.append(jnp.max(xm, axis=0, keepdims=True))
            x4 = jnp.concatenate(picked, axis=0)
            bids = jnp.concatenate(
                [
                    jnp.full((1, 1), bb_pref[s * TOPK + j], jnp.int32)
                    for j in range(TOPK)
                ],
                axis=0,
            )
            cols = (
                jax.lax.broadcasted_iota(jnp.int32, (TOPK, BLK), 1)
                + bids * BLK
            )
            x4 = jnp.where(cols < N, x4, NEG)
            bv, bi = _select_topk(x4, cols)
            cv = jnp.concatenate(
                [bv[q:q + 1, :] for q in range(TOPK)], axis=1
            )
            ci = jnp.concatenate(
                [bi[q:q + 1, :] for q in range(TOPK)], axis=1
            )
            nv, ni = _select_topk(cv, ci)
            rmask = jax.lax.broadcasted_iota(jnp.int32, (B, TOPK), 0) == s
            sv[...] = jnp.where(rmask, jnp.broadcast_to(nv, (B, TOPK)), sv[...])
            si[...] = jnp.where(rmask, jnp.broadcast_to(ni, (B, TOPK)), si[...])

        @pl.when(ph == 1)
        def _write():
            @pl.when(s == 0)
            def _finalize():
                v = sv[...]
                ix = si[...]
                rv = v / jnp.sum(v, axis=1, keepdims=True)
                srv[...] = rv
                p = ix + jax.lax.broadcasted_iota(jnp.int32, (B, TOPK), 0) * N
                score = jnp.log(rv + jnp.float32(1e-20)) + _gumbel_at(p)
                m = jnp.max(score, axis=1, keepdims=True)
                j2 = jax.lax.broadcasted_iota(jnp.int32, (B, TOPK), 1)
                jsel = jnp.min(
                    jnp.where(score == m, j2, BIGI), axis=1, keepdims=True
                )
                s_ref[...] = jnp.sum(
                    jnp.where(j2 == jsel, ix, 0), axis=1, keepdims=True
                )

            w = jnp.minimum(s, nblkw - 1)
            col = jax.lax.broadcasted_iota(jnp.int32, (B, BLKW), 1) + w * BLKW
            hit = jnp.any(si[...] // BLKW == w)

            @pl.when(hit)
            def _scatter_slab():
                acc = jnp.zeros((B, BLKW), jnp.float32)
                for jj in range(TOPK):
                    cj = si[:, pl.ds(jj, 1)]
                    vj = srv[:, pl.ds(jj, 1)]
                    acc = jnp.where(col == cj, vj, acc)
                out_ref[...] = acc

            @pl.when(jnp.logical_not(hit))
            def _zeros_slab():
                out_ref[...] = jnp.zeros((B, BLKW), jnp.float32)

    return body


def kernel(softmax):
    B, N = softmax.shape
    nblk = (N + BLK - 1) // BLK
    nsteps = (N + BLKF - 1) // BLKF
    nblkw = (N + BLKW - 1) // BLKW

    bb = pl.pallas_call(
        _make_blockmax_kernel(B, N, nblk, nsteps),
        grid=(nsteps,),
        in_specs=[pl.BlockSpec((B, BLKF), lambda i: (0, i))],
        out_specs=pl.BlockSpec((B, TOPK), lambda i: (0, 0)),
        out_shape=jax.ShapeDtypeStruct((B, TOPK), jnp.int32),
        scratch_shapes=[pltpu.VMEM((B, NBPAD), jnp.float32)],
    )(softmax)

    G = max(B, nblkw)
    cand_spec = lambda j: pl.BlockSpec(
        (8, BLK),
        lambda ph, s, bb, j=j: (
            (s // 8) * (1 - ph),
            bb[s * TOPK + j] * (1 - ph),
        ),
    )
    renorm, s2d = pl.pallas_call(
        _make_main_kernel(B, N, nblkw),
        grid_spec=pltpu.PrefetchScalarGridSpec(
            num_scalar_prefetch=1,
            grid=(2, G),
            in_specs=[cand_spec(j) for j in range(TOPK)],
            out_specs=[
                pl.BlockSpec(
                    (B, BLKW),
                    lambda ph, s, bb: (0, jnp.minimum(s, nblkw - 1) * ph),
                ),
                pl.BlockSpec((B, 1), lambda ph, s, bb: (0, 0)),
            ],
            scratch_shapes=[
                pltpu.VMEM((B, TOPK), jnp.float32),
                pltpu.VMEM((B, TOPK), jnp.int32),
                pltpu.VMEM((B, TOPK), jnp.float32),
            ],
        ),
        out_shape=[
            jax.ShapeDtypeStruct((B, N), jnp.float32),
            jax.ShapeDtypeStruct((B, 1), jnp.int32),
        ],
    )(bb.reshape(-1), softmax, softmax, softmax, softmax)

    return renorm, s2d.reshape(B)


# speedup vs baseline: 5.9217x; 1.2497x over previous
import numpy as np
import jax
import jax.numpy as jnp
from jax.experimental import pallas as pl
from jax.experimental.pallas import tpu as pltpu

TOPK = 4
BLKC = 2048
BLKF = 32768
BLKW = 32768
NBPAD = 512
RG = 8
RPS = 2
BIGI = np.int32(2**30)
NEG = np.float32(-np.inf)
TINY = np.float32(np.finfo(np.float32).tiny)


def _select_topk(vals, idx, k=TOPK):
    out_v, out_i = [], []
    for _ in range(k):
        m = jnp.max(vals, axis=1, keepdims=True)
        sel = jnp.min(jnp.where(vals == m, idx, BIGI), axis=1, keepdims=True)
        out_v.append(m)
        out_i.append(sel)
        vals = jnp.where(idx == sel, NEG, vals)
    return jnp.concatenate(out_v, axis=1), jnp.concatenate(out_i, axis=1)


def _threefry_bits(p_u32):
    k0 = jnp.uint32(0)
    k1 = jnp.uint32(42)
    k2 = k0 ^ k1 ^ jnp.uint32(0x1BD11BDA)
    ks = [k0, k1, k2]
    rot0 = (13, 15, 26, 6)
    rot1 = (17, 29, 16, 24)

    def rotl(x, d):
        return (x << jnp.uint32(d)) | (x >> jnp.uint32(32 - d))

    x0 = jnp.zeros_like(p_u32) + k0
    x1 = p_u32 + k1
    for r in range(5):
        for d in (rot0 if r % 2 == 0 else rot1):
            x0 = x0 + x1
            x1 = rotl(x1, d) ^ x0
        x0 = x0 + ks[(r + 1) % 3]
        x1 = x1 + ks[(r + 2) % 3] + jnp.uint32(r + 1)
    return x0 ^ x1


def _gumbel_at(p_i32):
    bits = _threefry_bits(p_i32.astype(jnp.uint32))
    fb = (bits >> jnp.uint32(9)) | jnp.uint32(0x3F800000)
    f = jax.lax.bitcast_convert_type(fb, jnp.float32) - jnp.float32(1.0)
    u = jnp.maximum(TINY, f + TINY)
    return -jnp.log(-jnp.log(u))


def _make_blockmax_kernel(B, N, nblk, nsteps):
    nsub = BLKF // BLKC

    def body(x_ref, bb_ref, bm):
        step = pl.program_id(0)

        @pl.when(step == 0)
        def _init():
            bm[...] = jnp.full((B, NBPAD), NEG, jnp.float32)

        lane = jax.lax.broadcasted_iota(jnp.int32, (B, NBPAD), 1)

        def merge(x):
            for q in range(nsub):
                k = step * nsub + q
                m = jnp.max(
                    x[:, q * BLKC:(q + 1) * BLKC], axis=1, keepdims=True
                )
                bm[...] = jnp.where(
                    (lane == k) & (k < nblk),
                    jnp.broadcast_to(m, (B, NBPAD)),
                    bm[...],
                )

        merge(x_ref[...])

        @pl.when(step == nsteps - 1)
        def _fix_tail_and_select():
            col = (
                jax.lax.broadcasted_iota(jnp.int32, (B, BLKF), 1) + step * BLKF
            )
            merge(jnp.where(col < N, x_ref[...], NEG))
            bids = jax.lax.broadcasted_iota(jnp.int32, (B, NBPAD), 1)
            _, bb = _select_topk(bm[...], bids)
            bb_ref[...] = bb

    return body


def _make_main_kernel(B, N, nblkw):
    nops = RPS * TOPK

    def body(bb_pref, *refs):
        xrefs = refs[:nops]
        out_ref, s_ref, sv, si, srv = refs[nops:]
        ph = pl.program_id(0)
        s = pl.program_id(1)

        @pl.when((ph == 0) & (s < B // RPS))
        def _scan_rows():
            sub = jax.lax.broadcasted_iota(jnp.int32, (RG, BLKC), 0)
            for t in range(RPS):
                r = s * RPS + t
                picked = []
                for j in range(TOPK):
                    xref = xrefs[t * TOPK + j]
                    xm = jnp.where(sub == r % RG, xref[...], NEG)
                    picked.append(jnp.max(xm, axis=0, keepdims=True))
                x4 = jnp.concatenate(picked, axis=0)
                bids = jnp.concatenate(
                    [
                        jnp.full((1, 1), bb_pref[r * TOPK + j], jnp.int32)
                        for j in range(TOPK)
                    ],
                    axis=0,
                )
                cols = (
                    jax.lax.broadcasted_iota(jnp.int32, (TOPK, BLKC), 1)
                    + bids * BLKC
                )
                x4 = jnp.where(cols < N, x4, NEG)
                bv, bi = _select_topk(x4, cols)
                cv = jnp.concatenate(
                    [bv[q:q + 1, :] for q in range(TOPK)], axis=1
                )
                ci = jnp.concatenate(
                    [bi[q:q + 1, :] for q in range(TOPK)], axis=1
                )
                nv, ni = _select_topk(cv, ci)
                rmask = (
                    jax.lax.broadcasted_iota(jnp.int32, (B, TOPK), 0) == r
                )
                sv[...] = jnp.where(
                    rmask, jnp.broadcast_to(nv, (B, TOPK)), sv[...]
                )
                si[...] = jnp.where(
                    rmask, jnp.broadcast_to(ni, (B, TOPK)), si[...]
                )

        @pl.when(ph == 1)
        def _write():
            @pl.when(s == 0)
            def _finalize():
                v = sv[...]
                ix = si[...]
                rv = v / jnp.sum(v, axis=1, keepdims=True)
                srv[...] = rv
                p = ix + jax.lax.broadcasted_iota(jnp.int32, (B, TOPK), 0) * N
                score = jnp.log(rv + jnp.float32(1e-20)) + _gumbel_at(p)
                m = jnp.max(score, axis=1, keepdims=True)
                j2 = jax.lax.broadcasted_iota(jnp.int32, (B, TOPK), 1)
                jsel = jnp.min(
                    jnp.where(score == m, j2, BIGI), axis=1, keepdims=True
                )
                s_ref[...] = jnp.sum(
                    jnp.where(j2 == jsel, ix, 0), axis=1, keepdims=True
                )

            w = jnp.minimum(s, nblkw - 1)
            col = jax.lax.broadcasted_iota(jnp.int32, (B, BLKW), 1) + w * BLKW
            hit = jnp.any(si[...] // BLKW == w)

            @pl.when(hit)
            def _scatter_slab():
                acc = jnp.zeros((B, BLKW), jnp.float32)
                for jj in range(TOPK):
                    cj = si[:, pl.ds(jj, 1)]
                    vj = srv[:, pl.ds(jj, 1)]
                    acc = jnp.where(col == cj, vj, acc)
                out_ref[...] = acc

            @pl.when(jnp.logical_not(hit))
            def _zeros_slab():
                out_ref[...] = jnp.zeros((B, BLKW), jnp.float32)

    return body


def kernel(softmax):
    B, N = softmax.shape
    nblk = (N + BLKC - 1) // BLKC
    nsteps = (N + BLKF - 1) // BLKF
    nblkw = (N + BLKW - 1) // BLKW
    assert nblk <= NBPAD and B % RPS == 0

    bb = pl.pallas_call(
        _make_blockmax_kernel(B, N, nblk, nsteps),
        grid=(nsteps,),
        in_specs=[pl.BlockSpec((B, BLKF), lambda i: (0, i))],
        out_specs=pl.BlockSpec((B, TOPK), lambda i: (0, 0)),
        out_shape=jax.ShapeDtypeStruct((B, TOPK), jnp.int32),
        scratch_shapes=[pltpu.VMEM((B, NBPAD), jnp.float32)],
    )(softmax)

    G = max(B // RPS, nblkw)

    def cand_spec(t, j):
        return pl.BlockSpec(
            (RG, BLKC),
            lambda ph, s, bb, t=t, j=j: (
                (jnp.minimum(s * RPS + t, B - 1) // RG) * (1 - ph),
                bb[jnp.minimum((s * RPS + t) * TOPK + j, B * TOPK - 1)]
                * (1 - ph),
            ),
        )

    renorm, s2d = pl.pallas_call(
        _make_main_kernel(B, N, nblkw),
        grid_spec=pltpu.PrefetchScalarGridSpec(
            num_scalar_prefetch=1,
            grid=(2, G),
            in_specs=[
                cand_spec(t, j) for t in range(RPS) for j in range(TOPK)
            ],
            out_specs=[
                pl.BlockSpec(
                    (B, BLKW),
                    lambda ph, s, bb: (0, jnp.minimum(s, nblkw - 1) * ph),
                ),
                pl.BlockSpec((B, 1), lambda ph, s, bb: (0, 0)),
            ],
            scratch_shapes=[
                pltpu.VMEM((B, TOPK), jnp.float32),
                pltpu.VMEM((B, TOPK), jnp.int32),
                pltpu.VMEM((B, TOPK), jnp.float32),
            ],
        ),
        out_shape=[
            jax.ShapeDtypeStruct((B, N), jnp.float32),
            jax.ShapeDtypeStruct((B, 1), jnp.int32),
        ],
    )(bb.reshape(-1), *([softmax] * (RPS * TOPK)))

    return renorm, s2d.reshape(B)


# speedup vs baseline: 6.3854x; 1.0783x over previous
import numpy as np
import jax
import jax.numpy as jnp
from jax.experimental import pallas as pl
from jax.experimental.pallas import tpu as pltpu

TOPK = 4
BLKC = 2048
BLKF = 32768
BLKW = 65536
NBPAD = 512
RG = 8
RPS = 2
BIGI = np.int32(2**30)
NEG = np.float32(-np.inf)
TINY = np.float32(np.finfo(np.float32).tiny)


def _select_topk(vals, idx, k=TOPK):
    out_v, out_i = [], []
    for _ in range(k):
        m = jnp.max(vals, axis=1, keepdims=True)
        sel = jnp.min(jnp.where(vals == m, idx, BIGI), axis=1, keepdims=True)
        out_v.append(m)
        out_i.append(sel)
        vals = jnp.where(idx == sel, NEG, vals)
    return jnp.concatenate(out_v, axis=1), jnp.concatenate(out_i, axis=1)


def _threefry_bits(p_u32):
    k0 = jnp.uint32(0)
    k1 = jnp.uint32(42)
    k2 = k0 ^ k1 ^ jnp.uint32(0x1BD11BDA)
    ks = [k0, k1, k2]
    rot0 = (13, 15, 26, 6)
    rot1 = (17, 29, 16, 24)

    def rotl(x, d):
        return (x << jnp.uint32(d)) | (x >> jnp.uint32(32 - d))

    x0 = jnp.zeros_like(p_u32) + k0
    x1 = p_u32 + k1
    for r in range(5):
        for d in (rot0 if r % 2 == 0 else rot1):
            x0 = x0 + x1
            x1 = rotl(x1, d) ^ x0
        x0 = x0 + ks[(r + 1) % 3]
        x1 = x1 + ks[(r + 2) % 3] + jnp.uint32(r + 1)
    return x0 ^ x1


def _gumbel_at(p_i32):
    bits = _threefry_bits(p_i32.astype(jnp.uint32))
    fb = (bits >> jnp.uint32(9)) | jnp.uint32(0x3F800000)
    f = jax.lax.bitcast_convert_type(fb, jnp.float32) - jnp.float32(1.0)
    u = jnp.maximum(TINY, f + TINY)
    return -jnp.log(-jnp.log(u))


def _make_blockmax_kernel(B, N, nblk, nsteps):
    nsub = BLKF // BLKC

    def body(x_ref, bb_ref, bm):
        step = pl.program_id(0)

        @pl.when(step == 0)
        def _init():
            bm[...] = jnp.full((B, NBPAD), NEG, jnp.float32)

        lane = jax.lax.broadcasted_iota(jnp.int32, (B, NBPAD), 1)

        def merge(x):
            for q in range(nsub):
                k = step * nsub + q
                m = jnp.max(
                    x[:, q * BLKC:(q + 1) * BLKC], axis=1, keepdims=True
                )
                bm[...] = jnp.where(
                    (lane == k) & (k < nblk),
                    jnp.broadcast_to(m, (B, NBPAD)),
                    bm[...],
                )

        merge(x_ref[...])

        @pl.when(step == nsteps - 1)
        def _fix_tail_and_select():
            col = (
                jax.lax.broadcasted_iota(jnp.int32, (B, BLKF), 1) + step * BLKF
            )
            merge(jnp.where(col < N, x_ref[...], NEG))
            bids = jax.lax.broadcasted_iota(jnp.int32, (B, NBPAD), 1)
            _, bb = _select_topk(bm[...], bids)
            bb_ref[...] = bb

    return body


def _make_main_kernel(B, N, nblkw):
    nops = RPS * TOPK

    def body(bb_pref, *refs):
        xrefs = refs[:nops]
        out_ref, s_ref, sv, si, srv = refs[nops:]
        ph = pl.program_id(0)
        s = pl.program_id(1)

        @pl.when((ph == 0) & (s < B // RPS))
        def _scan_rows():
            sub = jax.lax.broadcasted_iota(jnp.int32, (RG, BLKC), 0)
            for t in range(RPS):
                r = s * RPS + t
                picked = []
                for j in range(TOPK):
                    xref = xrefs[t * TOPK + j]
                    xm = jnp.where(sub == r % RG, xref[...], NEG)
                    picked.append(jnp.max(xm, axis=0, keepdims=True))
                x4 = jnp.concatenate(picked, axis=0)
                bids = jnp.concatenate(
                    [
                        jnp.full((1, 1), bb_pref[r * TOPK + j], jnp.int32)
                        for j in range(TOPK)
                    ],
                    axis=0,
                )
                cols = (
                    jax.lax.broadcasted_iota(jnp.int32, (TOPK, BLKC), 1)
                    + bids * BLKC
                )
                x4 = jnp.where(cols < N, x4, NEG)
                bv, bi = _select_topk(x4, cols)
                cv = jnp.concatenate(
                    [bv[q:q + 1, :] for q in range(TOPK)], axis=1
                )
                ci = jnp.concatenate(
                    [bi[q:q + 1, :] for q in range(TOPK)], axis=1
                )
                nv, ni = _select_topk(cv, ci)
                rmask = (
                    jax.lax.broadcasted_iota(jnp.int32, (B, TOPK), 0) == r
                )
                sv[...] = jnp.where(
                    rmask, jnp.broadcast_to(nv, (B, TOPK)), sv[...]
                )
                si[...] = jnp.where(
                    rmask, jnp.broadcast_to(ni, (B, TOPK)), si[...]
                )

        @pl.when(ph == 1)
        def _write():
            @pl.when(s == 0)
            def _finalize():
                v = sv[...]
                ix = si[...]
                rv = v / jnp.sum(v, axis=1, keepdims=True)
                srv[...] = rv
                p = ix + jax.lax.broadcasted_iota(jnp.int32, (B, TOPK), 0) * N
                score = jnp.log(rv + jnp.float32(1e-20)) + _gumbel_at(p)
                m = jnp.max(score, axis=1, keepdims=True)
                j2 = jax.lax.broadcasted_iota(jnp.int32, (B, TOPK), 1)
                jsel = jnp.min(
                    jnp.where(score == m, j2, BIGI), axis=1, keepdims=True
                )
                s_ref[...] = jnp.sum(
                    jnp.where(j2 == jsel, ix, 0), axis=1, keepdims=True
                )

            w = jnp.minimum(s, nblkw - 1)
            col = jax.lax.broadcasted_iota(jnp.int32, (B, BLKW), 1) + w * BLKW
            acc = jnp.zeros((B, BLKW), jnp.float32)
            for jj in range(TOPK):
                cj = si[:, pl.ds(jj, 1)]
                vj = srv[:, pl.ds(jj, 1)]
                acc = jnp.where(col == cj, vj, acc)
            out_ref[...] = acc

    return body


def kernel(softmax):
    B, N = softmax.shape
    nblk = (N + BLKC - 1) // BLKC
    nsteps = (N + BLKF - 1) // BLKF
    nblkw = (N + BLKW - 1) // BLKW
    assert nblk <= NBPAD and B % RPS == 0

    bb = pl.pallas_call(
        _make_blockmax_kernel(B, N, nblk, nsteps),
        grid=(nsteps,),
        in_specs=[pl.BlockSpec((B, BLKF), lambda i: (0, i))],
        out_specs=pl.BlockSpec((B, TOPK), lambda i: (0, 0)),
        out_shape=jax.ShapeDtypeStruct((B, TOPK), jnp.int32),
        scratch_shapes=[pltpu.VMEM((B, NBPAD), jnp.float32)],
    )(softmax)

    G = max(B // RPS, nblkw)

    def cand_spec(t, j):
        return pl.BlockSpec(
            (RG, BLKC),
            lambda ph, s, bb, t=t, j=j: (
                (jnp.minimum(s * RPS + t, B - 1) // RG) * (1 - ph),
                bb[jnp.minimum((s * RPS + t) * TOPK + j, B * TOPK - 1)]
                * (1 - ph),
            ),
        )

    renorm, s2d = pl.pallas_call(
        _make_main_kernel(B, N, nblkw),
        grid_spec=pltpu.PrefetchScalarGridSpec(
            num_scalar_prefetch=1,
            grid=(2, G),
            in_specs=[
                cand_spec(t, j) for t in range(RPS) for j in range(TOPK)
            ],
            out_specs=[
                pl.BlockSpec(
                    (B, BLKW),
                    lambda ph, s, bb: (0, jnp.minimum(s, nblkw - 1) * ph),
                ),
                pl.BlockSpec((B, 1), lambda ph, s, bb: (0, 0)),
            ],
            scratch_shapes=[
                pltpu.VMEM((B, TOPK), jnp.float32),
                pltpu.VMEM((B, TOPK), jnp.int32),
                pltpu.VMEM((B, TOPK), jnp.float32),
            ],
        ),
        out_shape=[
            jax.ShapeDtypeStruct((B, N), jnp.float32),
            jax.ShapeDtypeStruct((B, 1), jnp.int32),
        ],
    )(bb.reshape(-1), *([softmax] * (RPS * TOPK)))

    return renorm, s2d.reshape(B)


# speedup vs baseline: 6.6512x; 1.0416x over previous
import numpy as np
import jax
import jax.numpy as jnp
from jax.experimental import pallas as pl
from jax.experimental.pallas import tpu as pltpu

TOPK = 4
BLKC = 2048
BLKF = 32768
BLKW = 65536
NBPAD = 512
RG = 8
RPS = 4
BIGI = np.int32(2**30)
NEG = np.float32(-np.inf)
TINY = np.float32(np.finfo(np.float32).tiny)


def _select_topk(vals, idx, k=TOPK):
    out_v, out_i = [], []
    for _ in range(k):
        m = jnp.max(vals, axis=1, keepdims=True)
        sel = jnp.min(jnp.where(vals == m, idx, BIGI), axis=1, keepdims=True)
        out_v.append(m)
        out_i.append(sel)
        vals = jnp.where(idx == sel, NEG, vals)
    return jnp.concatenate(out_v, axis=1), jnp.concatenate(out_i, axis=1)


def _threefry_bits(p_u32):
    k0 = jnp.uint32(0)
    k1 = jnp.uint32(42)
    k2 = k0 ^ k1 ^ jnp.uint32(0x1BD11BDA)
    ks = [k0, k1, k2]
    rot0 = (13, 15, 26, 6)
    rot1 = (17, 29, 16, 24)

    def rotl(x, d):
        return (x << jnp.uint32(d)) | (x >> jnp.uint32(32 - d))

    x0 = jnp.zeros_like(p_u32) + k0
    x1 = p_u32 + k1
    for r in range(5):
        for d in (rot0 if r % 2 == 0 else rot1):
            x0 = x0 + x1
            x1 = rotl(x1, d) ^ x0
        x0 = x0 + ks[(r + 1) % 3]
        x1 = x1 + ks[(r + 2) % 3] + jnp.uint32(r + 1)
    return x0 ^ x1


def _gumbel_at(p_i32):
    bits = _threefry_bits(p_i32.astype(jnp.uint32))
    fb = (bits >> jnp.uint32(9)) | jnp.uint32(0x3F800000)
    f = jax.lax.bitcast_convert_type(fb, jnp.float32) - jnp.float32(1.0)
    u = jnp.maximum(TINY, f + TINY)
    return -jnp.log(-jnp.log(u))


def _make_blockmax_kernel(B, N, nblk, nsteps):
    nsub = BLKF // BLKC

    def body(x_ref, bb_ref, bm):
        step = pl.program_id(0)

        @pl.when(step == 0)
        def _init():
            bm[...] = jnp.full((B, NBPAD), NEG, jnp.float32)

        lane = jax.lax.broadcasted_iota(jnp.int32, (B, NBPAD), 1)

        def merge(x):
            for q in range(nsub):
                k = step * nsub + q
                m = jnp.max(
                    x[:, q * BLKC:(q + 1) * BLKC], axis=1, keepdims=True
                )
                bm[...] = jnp.where(
                    (lane == k) & (k < nblk),
                    jnp.broadcast_to(m, (B, NBPAD)),
                    bm[...],
                )

        merge(x_ref[...])

        @pl.when(step == nsteps - 1)
        def _fix_tail_and_select():
            col = (
                jax.lax.broadcasted_iota(jnp.int32, (B, BLKF), 1) + step * BLKF
            )
            merge(jnp.where(col < N, x_ref[...], NEG))
            bids = jax.lax.broadcasted_iota(jnp.int32, (B, NBPAD), 1)
            _, bb = _select_topk(bm[...], bids)
            bb_ref[...] = bb

    return body


def _make_main_kernel(B, N, nblkw):
    nops = RPS * TOPK

    def body(bb_pref, *refs):
        xrefs = refs[:nops]
        out_ref, s_ref, sv, si, srv = refs[nops:]
        ph = pl.program_id(0)
        s = pl.program_id(1)

        @pl.when((ph == 0) & (s < B // RPS))
        def _scan_rows():
            sub = jax.lax.broadcasted_iota(jnp.int32, (RG, BLKC), 0)
            for t in range(RPS):
                r = s * RPS + t
                picked = []
                for j in range(TOPK):
                    xref = xrefs[t * TOPK + j]
                    xm = jnp.where(sub == r % RG, xref[...], NEG)
                    picked.append(jnp.max(xm, axis=0, keepdims=True))
                x4 = jnp.concatenate(picked, axis=0)
                bids = jnp.concatenate(
                    [
                        jnp.full((1, 1), bb_pref[r * TOPK + j], jnp.int32)
                        for j in range(TOPK)
                    ],
                    axis=0,
                )
                cols = (
                    jax.lax.broadcasted_iota(jnp.int32, (TOPK, BLKC), 1)
                    + bids * BLKC
                )
                x4 = jnp.where(cols < N, x4, NEG)
                bv, bi = _select_topk(x4, cols)
                cv = jnp.concatenate(
                    [bv[q:q + 1, :] for q in range(TOPK)], axis=1
                )
                ci = jnp.concatenate(
                    [bi[q:q + 1, :] for q in range(TOPK)], axis=1
                )
                nv, ni = _select_topk(cv, ci)
                rmask = (
                    jax.lax.broadcasted_iota(jnp.int32, (B, TOPK), 0) == r
                )
                sv[...] = jnp.where(
                    rmask, jnp.broadcast_to(nv, (B, TOPK)), sv[...]
                )
                si[...] = jnp.where(
                    rmask, jnp.broadcast_to(ni, (B, TOPK)), si[...]
                )

        @pl.when(ph == 1)
        def _write():
            @pl.when(s == 0)
            def _finalize():
                v = sv[...]
                ix = si[...]
                rv = v / jnp.sum(v, axis=1, keepdims=True)
                srv[...] = rv
                p = ix + jax.lax.broadcasted_iota(jnp.int32, (B, TOPK), 0) * N
                score = jnp.log(rv + jnp.float32(1e-20)) + _gumbel_at(p)
                m = jnp.max(score, axis=1, keepdims=True)
                j2 = jax.lax.broadcasted_iota(jnp.int32, (B, TOPK), 1)
                jsel = jnp.min(
                    jnp.where(score == m, j2, BIGI), axis=1, keepdims=True
                )
                s_ref[...] = jnp.sum(
                    jnp.where(j2 == jsel, ix, 0), axis=1, keepdims=True
                )

            w = jnp.minimum(s, nblkw - 1)
            col = jax.lax.broadcasted_iota(jnp.int32, (B, BLKW), 1) + w * BLKW
            acc = jnp.zeros((B, BLKW), jnp.float32)
            for jj in range(TOPK):
                cj = si[:, pl.ds(jj, 1)]
                vj = srv[:, pl.ds(jj, 1)]
                acc = jnp.where(col == cj, vj, acc)
            out_ref[...] = acc

    return body


def kernel(softmax):
    B, N = softmax.shape
    nblk = (N + BLKC - 1) // BLKC
    nsteps = (N + BLKF - 1) // BLKF
    nblkw = (N + BLKW - 1) // BLKW
    assert nblk <= NBPAD and B % RPS == 0

    bb = pl.pallas_call(
        _make_blockmax_kernel(B, N, nblk, nsteps),
        grid=(nsteps,),
        in_specs=[pl.BlockSpec((B, BLKF), lambda i: (0, i))],
        out_specs=pl.BlockSpec((B, TOPK), lambda i: (0, 0)),
        out_shape=jax.ShapeDtypeStruct((B, TOPK), jnp.int32),
        scratch_shapes=[pltpu.VMEM((B, NBPAD), jnp.float32)],
    )(softmax)

    G = max(B // RPS, nblkw)

    def cand_spec(t, j):
        return pl.BlockSpec(
            (RG, BLKC),
            lambda ph, s, bb, t=t, j=j: (
                (jnp.minimum(s * RPS + t, B - 1) // RG) * (1 - ph),
                bb[jnp.minimum((s * RPS + t) * TOPK + j, B * TOPK - 1)]
                * (1 - ph),
            ),
        )

    renorm, s2d = pl.pallas_call(
        _make_main_kernel(B, N, nblkw),
        grid_spec=pltpu.PrefetchScalarGridSpec(
            num_scalar_prefetch=1,
            grid=(2, G),
            in_specs=[
                cand_spec(t, j) for t in range(RPS) for j in range(TOPK)
            ],
            out_specs=[
                pl.BlockSpec(
                    (B, BLKW),
                    lambda ph, s, bb: (0, jnp.minimum(s, nblkw - 1) * ph),
                ),
                pl.BlockSpec((B, 1), lambda ph, s, bb: (0, 0)),
            ],
            scratch_shapes=[
                pltpu.VMEM((B, TOPK), jnp.float32),
                pltpu.VMEM((B, TOPK), jnp.int32),
                pltpu.VMEM((B, TOPK), jnp.float32),
            ],
        ),
        out_shape=[
            jax.ShapeDtypeStruct((B, N), jnp.float32),
            jax.ShapeDtypeStruct((B, 1), jnp.int32),
        ],
    )(bb.reshape(-1), *([softmax] * (RPS * TOPK)))

    return renorm, s2d.reshape(B)


# speedup vs baseline: 6.9025x; 1.0378x over previous
import numpy as np
import jax
import jax.numpy as jnp
from jax.experimental import pallas as pl
from jax.experimental.pallas import tpu as pltpu

TOPK = 4
BLKC = 2048
BLKF = 65536
BLKW = 65536
NBPAD = 512
RG = 8
RPS = 4
BIGI = np.int32(2**30)
NEG = np.float32(-np.inf)
TINY = np.float32(np.finfo(np.float32).tiny)


def _select_topk(vals, idx, k=TOPK):
    out_v, out_i = [], []
    for _ in range(k):
        m = jnp.max(vals, axis=1, keepdims=True)
        sel = jnp.min(jnp.where(vals == m, idx, BIGI), axis=1, keepdims=True)
        out_v.append(m)
        out_i.append(sel)
        vals = jnp.where(idx == sel, NEG, vals)
    return jnp.concatenate(out_v, axis=1), jnp.concatenate(out_i, axis=1)


def _threefry_bits(p_u32):
    k0 = jnp.uint32(0)
    k1 = jnp.uint32(42)
    k2 = k0 ^ k1 ^ jnp.uint32(0x1BD11BDA)
    ks = [k0, k1, k2]
    rot0 = (13, 15, 26, 6)
    rot1 = (17, 29, 16, 24)

    def rotl(x, d):
        return (x << jnp.uint32(d)) | (x >> jnp.uint32(32 - d))

    x0 = jnp.zeros_like(p_u32) + k0
    x1 = p_u32 + k1
    for r in range(5):
        for d in (rot0 if r % 2 == 0 else rot1):
            x0 = x0 + x1
            x1 = rotl(x1, d) ^ x0
        x0 = x0 + ks[(r + 1) % 3]
        x1 = x1 + ks[(r + 2) % 3] + jnp.uint32(r + 1)
    return x0 ^ x1


def _gumbel_at(p_i32):
    bits = _threefry_bits(p_i32.astype(jnp.uint32))
    fb = (bits >> jnp.uint32(9)) | jnp.uint32(0x3F800000)
    f = jax.lax.bitcast_convert_type(fb, jnp.float32) - jnp.float32(1.0)
    u = jnp.maximum(TINY, f + TINY)
    return -jnp.log(-jnp.log(u))


def _make_blockmax_kernel(B, N, nblk, nsteps):
    nsub = BLKF // BLKC

    def body(x_ref, bb_ref, bm):
        step = pl.program_id(0)

        @pl.when(step == 0)
        def _init():
            bm[...] = jnp.full((B, NBPAD), NEG, jnp.float32)

        lane = jax.lax.broadcasted_iota(jnp.int32, (B, NBPAD), 1)

        def merge(x):
            for q in range(nsub):
                k = step * nsub + q
                m = jnp.max(
                    x[:, q * BLKC:(q + 1) * BLKC], axis=1, keepdims=True
                )
                bm[...] = jnp.where(
                    (lane == k) & (k < nblk),
                    jnp.broadcast_to(m, (B, NBPAD)),
                    bm[...],
                )

        merge(x_ref[...])

        @pl.when(step == nsteps - 1)
        def _fix_tail_and_select():
            col = (
                jax.lax.broadcasted_iota(jnp.int32, (B, BLKF), 1) + step * BLKF
            )
            merge(jnp.where(col < N, x_ref[...], NEG))
            bids = jax.lax.broadcasted_iota(jnp.int32, (B, NBPAD), 1)
            _, bb = _select_topk(bm[...], bids)
            bb_ref[...] = bb

    return body


def _make_main_kernel(B, N, nblkw):
    nops = RPS * TOPK

    def body(bb_pref, *refs):
        xrefs = refs[:nops]
        out_ref, s_ref, sv, si, srv = refs[nops:]
        ph = pl.program_id(0)
        s = pl.program_id(1)

        @pl.when((ph == 0) & (s < B // RPS))
        def _scan_rows():
            sub = jax.lax.broadcasted_iota(jnp.int32, (RG, BLKC), 0)
            for t in range(RPS):
                r = s * RPS + t
                picked = []
                for j in range(TOPK):
                    xref = xrefs[t * TOPK + j]
                    xm = jnp.where(sub == r % RG, xref[...], NEG)
                    picked.append(jnp.max(xm, axis=0, keepdims=True))
                x4 = jnp.concatenate(picked, axis=0)
                bids = jnp.concatenate(
                    [
                        jnp.full((1, 1), bb_pref[r * TOPK + j], jnp.int32)
                        for j in range(TOPK)
                    ],
                    axis=0,
                )
                cols = (
                    jax.lax.broadcasted_iota(jnp.int32, (TOPK, BLKC), 1)
                    + bids * BLKC
                )
                x4 = jnp.where(cols < N, x4, NEG)
                bv, bi = _select_topk(x4, cols)
                cv = jnp.concatenate(
                    [bv[q:q + 1, :] for q in range(TOPK)], axis=1
                )
                ci = jnp.concatenate(
                    [bi[q:q + 1, :] for q in range(TOPK)], axis=1
                )
                nv, ni = _select_topk(cv, ci)
                rmask = (
                    jax.lax.broadcasted_iota(jnp.int32, (B, TOPK), 0) == r
                )
                sv[...] = jnp.where(
                    rmask, jnp.broadcast_to(nv, (B, TOPK)), sv[...]
                )
                si[...] = jnp.where(
                    rmask, jnp.broadcast_to(ni, (B, TOPK)), si[...]
                )

        @pl.when(ph == 1)
        def _write():
            @pl.when(s == 0)
            def _finalize():
                v = sv[...]
                ix = si[...]
                rv = v / jnp.sum(v, axis=1, keepdims=True)
                srv[...] = rv
                p = ix + jax.lax.broadcasted_iota(jnp.int32, (B, TOPK), 0) * N
                score = jnp.log(rv + jnp.float32(1e-20)) + _gumbel_at(p)
                m = jnp.max(score, axis=1, keepdims=True)
                j2 = jax.lax.broadcasted_iota(jnp.int32, (B, TOPK), 1)
                jsel = jnp.min(
                    jnp.where(score == m, j2, BIGI), axis=1, keepdims=True
                )
                s_ref[...] = jnp.sum(
                    jnp.where(j2 == jsel, ix, 0), axis=1, keepdims=True
                )

            w = jnp.minimum(s, nblkw - 1)
            col = jax.lax.broadcasted_iota(jnp.int32, (B, BLKW), 1)
            acc = jnp.zeros((B, BLKW), jnp.float32)
            for jj in range(TOPK):
                cj = si[:, pl.ds(jj, 1)] - w * BLKW
                vj = srv[:, pl.ds(jj, 1)]
                acc = jnp.where(col == cj, vj, acc)
            out_ref[...] = acc

    return body


def kernel(softmax):
    B, N = softmax.shape
    nblk = (N + BLKC - 1) // BLKC
    nsteps = (N + BLKF - 1) // BLKF
    nblkw = (N + BLKW - 1) // BLKW
    assert nblk <= NBPAD and B % RPS == 0

    bb = pl.pallas_call(
        _make_blockmax_kernel(B, N, nblk, nsteps),
        grid=(nsteps,),
        in_specs=[pl.BlockSpec((B, BLKF), lambda i: (0, i))],
        out_specs=pl.BlockSpec((B, TOPK), lambda i: (0, 0)),
        out_shape=jax.ShapeDtypeStruct((B, TOPK), jnp.int32),
        scratch_shapes=[pltpu.VMEM((B, NBPAD), jnp.float32)],
    )(softmax)

    G = max(B // RPS, nblkw)

    def cand_spec(t, j):
        return pl.BlockSpec(
            (RG, BLKC),
            lambda ph, s, bb, t=t, j=j: (
                (jnp.minimum(s * RPS + t, B - 1) // RG) * (1 - ph),
                bb[jnp.minimum((s * RPS + t) * TOPK + j, B * TOPK - 1)]
                * (1 - ph),
            ),
        )

    renorm, s2d = pl.pallas_call(
        _make_main_kernel(B, N, nblkw),
        grid_spec=pltpu.PrefetchScalarGridSpec(
            num_scalar_prefetch=1,
            grid=(2, G),
            in_specs=[
                cand_spec(t, j) for t in range(RPS) for j in range(TOPK)
            ],
            out_specs=[
                pl.BlockSpec(
                    (B, BLKW),
                    lambda ph, s, bb: (0, jnp.minimum(s, nblkw - 1) * ph),
                ),
                pl.BlockSpec((B, 1), lambda ph, s, bb: (0, 0)),
            ],
            scratch_shapes=[
                pltpu.VMEM((B, TOPK), jnp.float32),
                pltpu.VMEM((B, TOPK), jnp.int32),
                pltpu.VMEM((B, TOPK), jnp.float32),
            ],
        ),
        out_shape=[
            jax.ShapeDtypeStruct((B, N), jnp.float32),
            jax.ShapeDtypeStruct((B, 1), jnp.int32),
        ],
    )(bb.reshape(-1), *([softmax] * (RPS * TOPK)))

    return renorm, s2d.reshape(B)
